# Initial kernel scaffold; baseline (speedup 1.0000x reference)
import jax
import jax.numpy as jnp
from jax.experimental import pallas as pl


def kernel(x_static_node, x_static_graph, edge_index, edge_weight, batch, target_index, W1, b1, W2, b2, Wg1, bg1, Wg2, bg2, Wpre, bpre, Wcom, bcom, Wf1, bf1, Wf2, bf2):
    raise NotImplementedError("write your pallas kernel here")


# speedup vs baseline: 3.2838x; 3.2838x over previous
import functools

import jax
import jax.numpy as jnp
from jax import lax
from jax.experimental import pallas as pl
from jax.experimental.pallas import tpu as pltpu
from jax.experimental.pallas import tpu_sc as plsc

N_NODES = 10000
N_PAD = 10240
B_GRAPHS = 256
D_NODE = 128
HID = 256
NC = 2
NS = 16
LANES = 16
CH = 128
BM = 512
N_ROW_BLK = N_PAD // BM


def _mesh():
    return plsc.VectorSubcoreMesh(
        core_axis_name="c", subcore_axis_name="s", num_cores=NC, num_subcores=NS)


def _make_deg_kernel(n_chunks_total):
    per_tile = n_chunks_total // (NC * NS)
    rounds = per_tile // 8

    @functools.partial(
        pl.kernel,
        out_type=jax.ShapeDtypeStruct((NC, NS, N_PAD), jnp.float32),
        mesh=_mesh(),
        compiler_params=pltpu.CompilerParams(needs_layout_passes=False),
        scratch_types=[
            pltpu.VMEM((8, CH), jnp.int32),
            pltpu.VMEM((8, CH), jnp.float32),
            pltpu.VMEM((N_PAD,), jnp.float32),
        ],
    )
    def deg_kernel(col2d, ew2d, out_hbm, cbuf, wbuf, hist):
        c = lax.axis_index("c")
        s = lax.axis_index("s")
        gwid = s * NC + c
        zero16 = jnp.zeros((16,), jnp.float32)

        def zrow(i, _):
            hist[pl.ds(16 * i, 16)] = zero16
            return 0
        lax.fori_loop(0, N_PAD // 16, zrow, 0)

        base = gwid * per_tile

        def body(r, _):
            pltpu.sync_copy(col2d.at[pl.ds(base + r * 8, 8)], cbuf)
            pltpu.sync_copy(ew2d.at[pl.ds(base + r * 8, 8)], wbuf)
            for a in range(8):
                for k in range(CH // 16):
                    c16 = cbuf[a, pl.ds(16 * k, 16)]
                    w16 = wbuf[a, pl.ds(16 * k, 16)]
                    plsc.addupdate_scatter(hist, [c16], w16)
            return 0
        lax.fori_loop(0, rounds, body, 0)
        pltpu.sync_copy(hist, out_hbm.at[c, s])

    return deg_kernel


def _make_agg_kernel(n_chunks_total):
    per_tile = n_chunks_total // NS
    rows_per_tile = N_PAD // NS

    @functools.partial(
        pl.kernel,
        out_type=jax.ShapeDtypeStruct((NC, N_PAD, 128), jnp.float32),
        mesh=_mesh(),
        compiler_params=pltpu.CompilerParams(needs_layout_passes=False),
        scratch_types=[
            pltpu.VMEM((8, CH), jnp.int32),
            pltpu.VMEM((8, CH), jnp.int32),
            pltpu.VMEM((8, CH), jnp.float32),
            pltpu.VMEM((8 * CH,), jnp.float32),
            pltpu.VMEM((8, CH), jnp.int32),
            pltpu.VMEM((CH, 128), jnp.float32),
            pltpu.VMEM((CH, 128), jnp.float32),
            pltpu.VMEM((N_PAD,), jnp.float32),
            pltpu.VMEM((1, 128), jnp.float32),
            pltpu.VMEM_SHARED((N_PAD, 128), jnp.float32),
            pltpu.SemaphoreType.DMA,
            pltpu.SemaphoreType.DMA,
        ],
    )
    def agg_kernel(xw_hbm, row2d, col2d, ew2d, dinv_hbm, bias_hbm, out_hbm,
                   rbuf, cbuf, wbuf, abuf, gbuf, msg0, msg1, dinvbuf,
                   bbuf, acc, sem0, sem1):
        c = lax.axis_index("c")
        s = lax.axis_index("s")
        zero16 = jnp.zeros((16,), jnp.float32)
        zi = jnp.zeros((16,), jnp.int32)

        pltpu.sync_copy(dinv_hbm, dinvbuf)
        pltpu.sync_copy(bias_hbm.at[c], bbuf)

        def zrow(i, _):
            for j in range(8):
                msg0[i, pl.ds(16 * j, 16)] = zero16
            return 0
        lax.fori_loop(0, CH, zrow, 0)
        for q in range(rows_per_tile // CH):
            pltpu.sync_copy(msg0, acc.at[pl.ds(s * rows_per_tile + q * CH, CH)])
        plsc.subcore_barrier()

        base = s * per_tile
        coff = c * N_PAD

        def chunk(r, _):
            pltpu.sync_copy(row2d.at[pl.ds(base + r * 8, 8)], rbuf)
            pltpu.sync_copy(col2d.at[pl.ds(base + r * 8, 8)], cbuf)
            pltpu.sync_copy(ew2d.at[pl.ds(base + r * 8, 8)], wbuf)
            for a in range(8):
                for k in range(CH // 16):
                    r16 = rbuf[a, pl.ds(16 * k, 16)]
                    w16 = wbuf[a, pl.ds(16 * k, 16)]
                    dv = plsc.load_gather(dinvbuf, [r16])
                    abuf[pl.ds(a * CH + 16 * k, 16)] = w16 * dv
                    gbuf[a, pl.ds(16 * k, 16)] = r16 + coff
            bufs = (msg0, msg1)
            sems = (sem0, sem1)
            descs = [None, None]
            descs[0] = pltpu.async_copy(xw_hbm.at[gbuf.at[0]], msg0, sem0)
            for a in range(8):
                cur = a % 2
                if a < 7:
                    nxt = (a + 1) % 2
                    descs[nxt] = pltpu.async_copy(
                        xw_hbm.at[gbuf.at[a + 1]], bufs[nxt], sems[nxt])
                descs[cur].wait()
                mbuf = bufs[cur]

                def scale(i, _, a=a, mbuf=mbuf):
                    al = plsc.load_gather(abuf, [zi + (a * CH) + i])
                    for j in range(8):
                        mbuf[i, pl.ds(16 * j, 16)] = (
                            mbuf[i, pl.ds(16 * j, 16)] * al)
                    return 0
                lax.fori_loop(0, CH, scale, 0, unroll=2)
                pltpu.sync_copy(mbuf, acc.at[cbuf.at[a]], add=True)
            return 0
        lax.fori_loop(0, per_tile // 8, chunk, 0)
        plsc.subcore_barrier()

        r0 = s * rows_per_tile
        obuf = msg0
        for q in range(rows_per_tile // CH):
            rq = r0 + q * CH
            pltpu.sync_copy(acc.at[pl.ds(rq, CH)], obuf)

            zi = jnp.zeros((16,), jnp.int32)

            def fin(i, _):
                v = rq + i
                dvs = plsc.load_gather(dinvbuf, [zi + v])
                for j in range(8):
                    b16 = bbuf[0, pl.ds(16 * j, 16)]
                    obuf[i, pl.ds(16 * j, 16)] = jnp.maximum(
                        obuf[i, pl.ds(16 * j, 16)] * dvs + b16, 0.0)
                return 0
            lax.fori_loop(0, CH, fin, 0)
            pltpu.sync_copy(obuf, out_hbm.at[c, pl.ds(rq, CH)])

    return agg_kernel


def _dinv_body(deg_ref, out_ref):
    d = jnp.sum(deg_ref[...], axis=0)
    out_ref[...] = jnp.where(d > 0, jax.lax.rsqrt(d), 0.0)


def _dinv(deg):
    return pl.pallas_call(
        _dinv_body,
        out_shape=jax.ShapeDtypeStruct((80, 128), jnp.float32),
    )(deg.reshape(NC * NS, 80, 128))


def _mm_body(n_parts, x_ref, w_ref, o_ref):
    acc = jnp.zeros((BM, 128), jnp.float32)
    for p in range(n_parts):
        acc = acc + jax.lax.dot(x_ref[p], w_ref[p],
                                preferred_element_type=jnp.float32)
    o_ref[...] = acc


def _mm(x3d, w3d):
    n_parts = x3d.shape[0]
    return pl.pallas_call(
        functools.partial(_mm_body, n_parts),
        grid=(N_ROW_BLK, 2),
        in_specs=[
            pl.BlockSpec((n_parts, BM, 128), lambda i, j: (0, i, 0)),
            pl.BlockSpec((n_parts, 128, 128), lambda i, j: (0, 0, j)),
        ],
        out_specs=pl.BlockSpec((BM, 128), lambda i, j: (j * N_ROW_BLK + i, 0)),
        out_shape=jax.ShapeDtypeStruct((2 * N_PAD, 128), jnp.float32),
    )(x3d, w3d)


def _pool_body(g_ref, b_ref, fs_ref, cnt_ref):
    j = pl.program_id(0)
    i = pl.program_id(1)
    bvals = b_ref[...]
    iota = jax.lax.broadcasted_iota(jnp.int32, (BM, B_GRAPHS), 1)
    P = (bvals == iota).astype(jnp.float32)
    dn = (((0,), (0,)), ((), ()))

    @pl.when(i == 0)
    def _():
        fs_ref[0] = jnp.zeros((B_GRAPHS, 128), jnp.float32)

    fs_ref[0] = fs_ref[0] + jax.lax.dot_general(
        P, g_ref[0], dn, preferred_element_type=jnp.float32)

    @pl.when(jnp.logical_and(i == 0, j == 0))
    def _():
        cnt_ref[...] = jnp.zeros((B_GRAPHS, 128), jnp.float32)

    @pl.when(j == 0)
    def _():
        ones = jnp.ones((BM, 128), jnp.float32)
        cnt_ref[...] = cnt_ref[...] + jax.lax.dot_general(
            P, ones, dn, preferred_element_type=jnp.float32)


def _pool(g, batch2d):
    return pl.pallas_call(
        _pool_body,
        grid=(2, N_ROW_BLK),
        in_specs=[
            pl.BlockSpec((1, BM, 128), lambda j, i: (j, i, 0)),
            pl.BlockSpec((BM, 1), lambda j, i: (i, 0)),
        ],
        out_specs=[
            pl.BlockSpec((1, B_GRAPHS, 128), lambda j, i: (j, 0, 0)),
            pl.BlockSpec((B_GRAPHS, 128), lambda j, i: (0, 0)),
        ],
        out_shape=[
            jax.ShapeDtypeStruct((2, B_GRAPHS, 128), jnp.float32),
            jax.ShapeDtypeStruct((B_GRAPHS, 128), jnp.float32),
        ],
    )(g, batch2d)


def _final_body(xn_ref, w1_ref, b1_ref, w2_ref, b2_ref, fs_ref, cnt_ref,
                wpre_ref, bpre_ref, wcom_ref, bcom_ref, wf1_ref, bf1_ref,
                wf2_ref, bf2_ref, out_ref, p_ref, f_ref):
    dot = functools.partial(jax.lax.dot, preferred_element_type=jnp.float32)
    h = jnp.maximum(dot(xn_ref[...], w1_ref[...]) + b1_ref[...], 0.0)
    pat = jnp.maximum(dot(h, w2_ref[...]) + b2_ref[...], 0.0)
    inv_cnt = 1.0 / jnp.maximum(cnt_ref[...], 1.0)
    fam_a = fs_ref[0] * inv_cnt
    fam_b = fs_ref[1] * inv_cnt
    fam = jnp.concatenate([fam_a, fam_b], axis=1)
    z = jnp.maximum(
        dot(pat, wpre_ref[0]) + dot(fam, wpre_ref[1]) + bpre_ref[...], 0.0)
    out_ref[...] = jax.nn.sigmoid(dot(z, wcom_ref[...]) + bcom_ref[...])
    p_ref[...] = jax.nn.sigmoid(dot(pat, wf1_ref[...]) + bf1_ref[...])
    f_ref[...] = jax.nn.sigmoid(dot(fam, wf2_ref[...]) + bf2_ref[...])


def _final(xn, W1, b1, W2, b2, fs, cnt, Wpre, bpre, Wcom_p, bcom_p,
           Wf1_p, bf1_p, Wf2_p, bf2_p):
    shp = jax.ShapeDtypeStruct((B_GRAPHS, 128), jnp.float32)
    return pl.pallas_call(
        _final_body,
        out_shape=[shp, shp, shp],
    )(xn, W1, b1, W2, b2, fs, cnt, Wpre, bpre, Wcom_p, bcom_p,
      Wf1_p, bf1_p, Wf2_p, bf2_p)


def kernel(x_static_node, x_static_graph, edge_index, edge_weight, batch,
           target_index, W1, b1, W2, b2, Wg1, bg1, Wg2, bg2, Wpre, bpre,
           Wcom, bcom, Wf1, bf1, Wf2, bf2):
    f32 = jnp.float32

    row = edge_index[0].astype(jnp.int32)
    col = edge_index[1].astype(jnp.int32)
    n_e = row.shape[0]
    loop = jnp.arange(N_NODES, dtype=jnp.int32)
    e_full = n_e + N_NODES
    n_chunks_total = -(-e_full // (NC * NS * 8 * CH)) * (NC * NS * 8)
    e_pad = n_chunks_total * CH - e_full
    row_f = jnp.concatenate([row, loop, jnp.zeros((e_pad,), jnp.int32)])
    col_f = jnp.concatenate([col, loop, jnp.zeros((e_pad,), jnp.int32)])
    ew_f = jnp.concatenate([edge_weight.astype(f32), jnp.ones((N_NODES,), f32),
                            jnp.zeros((e_pad,), f32)])
    row2d = row_f.reshape(n_chunks_total, CH)
    col2d = col_f.reshape(n_chunks_total, CH)
    ew2d = ew_f.reshape(n_chunks_total, CH)

    xg = jnp.pad(x_static_graph.astype(f32), ((0, N_PAD - N_NODES), (0, 0)))
    batch2d = jnp.pad(batch.astype(jnp.int32), (0, N_PAD - N_NODES),
                      constant_values=B_GRAPHS).reshape(N_PAD, 1)

    deg_k = _make_deg_kernel(n_chunks_total)
    agg_k = _make_agg_kernel(n_chunks_total)

    deg = deg_k(col2d, ew2d)
    dinv = _dinv(deg).reshape(N_PAD)

    xw1 = _mm(xg[None], Wg1.astype(f32).reshape(1, 128, HID))
    g1 = agg_k(xw1, row2d, col2d, ew2d, dinv,
               bg1.astype(f32).reshape(NC, 1, 128))

    xw2 = _mm(g1, Wg2.astype(f32).reshape(2, 128, HID))
    g2 = agg_k(xw2, row2d, col2d, ew2d, dinv,
               bg2.astype(f32).reshape(NC, 1, 128))

    fs, cnt = _pool(g2, batch2d)

    def pad_w(w):
        return jnp.pad(w.astype(f32), ((0, 0), (0, 128 - w.shape[1])))

    def pad_b(b):
        return jnp.pad(b.astype(f32), (0, 128 - b.shape[0])).reshape(1, 128)

    out, p, f = _final(
        x_static_node.astype(f32), W1.astype(f32), b1.astype(f32).reshape(1, HID),
        W2.astype(f32), b2.astype(f32).reshape(1, HID), fs, cnt,
        Wpre.astype(f32).reshape(2, HID, HID), bpre.astype(f32).reshape(1, HID),
        pad_w(Wcom), pad_b(bcom), pad_w(Wf1), pad_b(bf1), pad_w(Wf2), pad_b(bf2))
    return (out[:, :1], p[:, :1], f[:, :1])
